# baseline (speedup 1.0000x reference)
import functools

import jax
import jax.numpy as jnp
from jax import lax
from jax.experimental import pallas as pl
from jax.experimental.pallas import tpu as pltpu
from jax.experimental.pallas import tpu_sc as plsc

N = 10000
E = 320000
D = 128
NC = 2
NS = 16
NW = NC * NS
EPW = E // NW
CHUNK = 80
NCHUNK = EPW // CHUNK
SBC = 25
NSB = NCHUNK // SBC
NGBUF = 4
NSBUF = 2
NROUND = (SBC - 1) // NGBUF
NPAD = 10240
ROWS_PER_TILE = NPAD // NS
SNR = NPAD // D


_B1 = 2048


def _stage1_body(x_ref, w_ref, a2_ref, tab_ref, rd2_ref):
    s = jnp.dot(x_ref[...], w_ref[...], preferred_element_type=jnp.float32)
    rd = jax.nn.sigmoid(
        jnp.dot(s, a2_ref[...], preferred_element_type=jnp.float32))
    tab_ref[...] = (rd * s).astype(jnp.bfloat16)
    r_i = lax.broadcasted_iota(jnp.int32, (_B1 // D, _B1), 0)
    n_i = lax.broadcasted_iota(jnp.int32, (_B1 // D, _B1), 1)
    b1t = (n_i // D == r_i).astype(jnp.float32)
    n2 = lax.broadcasted_iota(jnp.int32, (_B1, D), 0)
    l2 = lax.broadcasted_iota(jnp.int32, (_B1, D), 1)
    sel = (n2 % D == l2).astype(jnp.float32)
    rd2_ref[...] = jnp.dot(b1t, rd * sel,
                           preferred_element_type=jnp.float32)


def _stage1(xp, W, a2):
    return pl.pallas_call(
        _stage1_body,
        grid=(NPAD // _B1,),
        in_specs=[
            pl.BlockSpec((_B1, D), lambda i: (i, 0)),
            pl.BlockSpec((D, D), lambda i: (0, 0)),
            pl.BlockSpec((D, 1), lambda i: (0, 0)),
        ],
        out_specs=[
            pl.BlockSpec((_B1, D), lambda i: (i, 0)),
            pl.BlockSpec((_B1 // D, D), lambda i: (i, 0)),
        ],
        out_shape=[
            jax.ShapeDtypeStruct((NPAD, D), jnp.bfloat16),
            jax.ShapeDtypeStruct((SNR, D), jnp.float32),
        ],
    )(xp, W, a2)


def _sc_body(tab_hbm, rd2_hbm, rc_hbm, val_hbm, out_hbm, sn_hbm,
             col_s, row_s, val_s, rowv0, rowv1,
             gb0, gb1, gb2, gb3, sb0, sb1,
             acc, st0, sg0, sg1, sg2, sg3, ss0, ss1):
    c = lax.axis_index("c")
    s = lax.axis_index("s")
    w = s * NC + c

    gbufs = [gb0, gb1, gb2, gb3]
    sbufs = [sb0, sb1]
    rowvs = [rowv0, rowv1]
    sg = [sg0, sg1, sg2, sg3]
    ss = [ss0, ss1]

    zf16 = jnp.zeros((16,), jnp.float32)

    def zero_body(i, carry):
        for j in range(D // 16):
            sb0[i, pl.ds(j * 16, 16)] = zf16
        return carry
    lax.fori_loop(0, CHUNK, zero_body, 0, unroll=False)
    for b in range(ROWS_PER_TILE // CHUNK):
        pltpu.async_copy(
            sb0, acc.at[pl.ds(s * ROWS_PER_TILE + b * CHUNK, CHUNK)], ss0)
    for b in range(ROWS_PER_TILE // CHUNK):
        pltpu.make_async_copy(
            sb0, acc.at[pl.ds(s * ROWS_PER_TILE + b * CHUNK, CHUNK)],
            ss0).wait()
    plsc.subcore_barrier()

    def start_gather(kk, b):
        pltpu.async_copy(tab_hbm.at[col_s.at[pl.ds(kk * CHUNK, CHUNK)]],
                         gbufs[b], sg[b])

    def wait_gather(kk, b):
        pltpu.make_async_copy(tab_hbm.at[col_s.at[pl.ds(kk * CHUNK, CHUNK)]],
                              gbufs[b], sg[b]).wait()

    def start_scatter(b2):
        pltpu.async_copy(sbufs[b2], acc.at[rowvs[b2]], ss[b2], add=True)

    def wait_scatter(b2):
        pltpu.make_async_copy(sbufs[b2], acc.at[rowvs[b2]], ss[b2]).wait()

    hi_mask = jnp.full((16,), -65536, jnp.int32)

    def process(kk, b, b2):
        gb = gbufs[b]
        sbuf = sbufs[b2]
        rowv = rowvs[b2]

        def group_body(g, carry2):
            base = kk * CHUNK + g * 16
            row16 = row_s[pl.ds(base, 16)]
            val16 = val_s[pl.ds(base, 16)]
            rowv[pl.ds(g * 16, 16)] = row16
            for l in range(16):
                v = val16[l]
                e = g * 16 + l
                for j in range(D // 32):
                    wv = gb[e, pl.ds(j * 16, 16)]
                    lo = plsc.bitcast(wv << 16, jnp.float32) * v
                    hi = plsc.bitcast(wv & hi_mask, jnp.float32) * v
                    sbuf[e, pl.ds(j * 32, 16)] = lo
                    sbuf[e, pl.ds(j * 32 + 16, 16)] = hi
            return carry2

        lax.fori_loop(0, CHUNK // 16, group_body, 0, unroll=False)

    def stage_cv(sb):
        base = w * EPW + sb * (SBC * CHUNK)
        pltpu.async_copy(rc_hbm.at[pl.ds(E + base, SBC * CHUNK)], col_s, st0)
        pltpu.async_copy(rc_hbm.at[pl.ds(base, SBC * CHUNK)], row_s, st0)
        pltpu.async_copy(val_hbm.at[pl.ds(base, SBC * CHUNK)], val_s, st0)
        pltpu.make_async_copy(rc_hbm.at[pl.ds(E + base, SBC * CHUNK)],
                              col_s, st0).wait()
        pltpu.make_async_copy(rc_hbm.at[pl.ds(base, SBC * CHUNK)],
                              row_s, st0).wait()
        pltpu.make_async_copy(val_hbm.at[pl.ds(base, SBC * CHUNK)],
                              val_s, st0).wait()

    def sb_body(sb, carry):
        stage_cv(sb)
        start_gather(0, 0)
        start_gather(1, 1)

        def round_body(r, carry2):
            for b in range(NGBUF):
                k = r * NGBUF + b
                b2 = b % NSBUF

                @pl.when(k >= 2)
                def _():
                    wait_scatter(b2)

                @pl.when(k <= SBC - 3)
                def _():
                    start_gather(k + 2, (b + 2) % NGBUF)

                wait_gather(k, b)
                process(k, b, b2)
                start_scatter(b2)
            return carry2

        lax.fori_loop(0, NROUND, round_body, 0, unroll=False)

        wait_scatter(0)
        wait_gather(SBC - 1, 0)
        process(SBC - 1, 0, 0)
        start_scatter(0)
        wait_scatter(1)
        wait_scatter(0)
        return carry

    lax.fori_loop(0, NSB, sb_body, 0, unroll=False)

    def zero_sn(i, carry):
        for j in range(D // 16):
            sb1[i, pl.ds(j * 16, 16)] = zf16
        return carry
    lax.fori_loop(0, CHUNK, zero_sn, 0, unroll=False)
    pltpu.sync_copy(rd2_hbm, sb0)

    def sb2_body(sb, carry):
        stage_cv(sb)

        def chunk2(kk, carry2):
            def group2(g, carry3):
                base = kk * CHUNK + g * 16
                col16 = col_s[pl.ds(base, 16)]
                row16 = row_s[pl.ds(base, 16)]
                val16 = val_s[pl.ds(base, 16)]
                rd16 = plsc.load_gather(
                    sb0,
                    [lax.shift_right_logical(col16, 7),
                     lax.bitwise_and(col16, 127)])
                plsc.addupdate_scatter(
                    sb1,
                    [lax.shift_right_logical(row16, 7),
                     lax.bitwise_and(row16, 127)],
                    val16 * rd16)
                return carry3

            return lax.fori_loop(0, CHUNK // 16, group2, carry2,
                                 unroll=False)

        lax.fori_loop(0, SBC, chunk2, 0, unroll=False)
        return carry

    lax.fori_loop(0, NSB, sb2_body, 0, unroll=False)

    pltpu.sync_copy(sb1, sn_hbm.at[w])
    plsc.subcore_barrier()

    pltpu.sync_copy(acc.at[pl.ds(s * ROWS_PER_TILE, ROWS_PER_TILE)],
                    out_hbm.at[c, pl.ds(s * ROWS_PER_TILE, ROWS_PER_TILE)])


_sc_kernel = functools.partial(
    pl.kernel,
    out_type=[
        jax.ShapeDtypeStruct((NC, NPAD, D), jnp.float32),
        jax.ShapeDtypeStruct((NW, SNR, D), jnp.float32),
    ],
    mesh=plsc.VectorSubcoreMesh(core_axis_name="c", subcore_axis_name="s"),
    compiler_params=pltpu.CompilerParams(needs_layout_passes=False,
                                        use_tc_tiling_on_sc=False),
    scratch_types=[
        pltpu.VMEM((SBC * CHUNK,), jnp.int32),
        pltpu.VMEM((SBC * CHUNK,), jnp.int32),
        pltpu.VMEM((SBC * CHUNK,), jnp.float32),
        pltpu.VMEM((CHUNK,), jnp.int32),
        pltpu.VMEM((CHUNK,), jnp.int32),
        pltpu.VMEM((CHUNK, D // 2), jnp.int32),
        pltpu.VMEM((CHUNK, D // 2), jnp.int32),
        pltpu.VMEM((CHUNK, D // 2), jnp.int32),
        pltpu.VMEM((CHUNK, D // 2), jnp.int32),
        pltpu.VMEM((CHUNK, D), jnp.float32),
        pltpu.VMEM((CHUNK, D), jnp.float32),
        pltpu.VMEM_SHARED((NPAD, D), jnp.float32),
        pltpu.SemaphoreType.DMA,
        pltpu.SemaphoreType.DMA,
        pltpu.SemaphoreType.DMA,
        pltpu.SemaphoreType.DMA,
        pltpu.SemaphoreType.DMA,
        pltpu.SemaphoreType.DMA,
        pltpu.SemaphoreType.DMA,
    ],
)(_sc_body)


_B3 = 2048


def _stage3_body(acc_ref, sn_ref, a1_ref, out_ref):
    a = acc_ref[0] + acc_ref[1]
    sn_sum = jnp.sum(sn_ref[...], axis=0)
    rows = lax.broadcasted_iota(jnp.int32, (_B3, _B3 // D), 0)
    cols = lax.broadcasted_iota(jnp.int32, (_B3, _B3 // D), 1)
    b1 = (rows // D == cols).astype(jnp.float32)
    tmp = jnp.dot(b1, sn_sum, preferred_element_type=jnp.float32)
    rows2 = lax.broadcasted_iota(jnp.int32, (_B3, D), 0)
    cols2 = lax.broadcasted_iota(jnp.int32, (_B3, D), 1)
    sel = (rows2 % D == cols2).astype(jnp.float32)
    snb = jnp.dot(tmp * sel, jnp.ones((D, D), jnp.float32),
                  preferred_element_type=jnp.float32)
    p2 = lax.broadcasted_iota(jnp.int32, (D, D), 0)
    f2 = lax.broadcasted_iota(jnp.int32, (D, D), 1)
    pi = 32 * (f2 // 32) + (f2 % 2) * 16 + (f2 % 32) // 2
    m = (p2 == pi).astype(jnp.float32)
    a1p = jnp.dot(m, a1_ref[...], preferred_element_type=jnp.float32)
    out = a / snb
    l_d = jax.nn.softplus(
        jnp.dot(out, a1p, preferred_element_type=jnp.float32))
    out = jnp.maximum(out, 0.0) + l_d * jnp.minimum(out, 0.0)
    out = jnp.where(out > 0, out, jnp.exp(jnp.minimum(out, 0.0)) - 1.0)
    out_ref[...] = jnp.dot(out, m, preferred_element_type=jnp.float32)


def _stage3(acc, sn, a1):
    return pl.pallas_call(
        _stage3_body,
        grid=(NPAD // _B3,),
        in_specs=[
            pl.BlockSpec((NC, _B3, D), lambda i: (0, i, 0)),
            pl.BlockSpec((NW, _B3 // D, D), lambda i: (0, i, 0)),
            pl.BlockSpec((D, 1), lambda i: (0, 0)),
        ],
        out_specs=pl.BlockSpec((_B3, D), lambda i: (i, 0)),
        out_shape=jax.ShapeDtypeStruct((NPAD, D), jnp.float32),
    )(acc, sn, a1)


def kernel(x, adj_indices, adj_values, W, a1, a2):
    rc = adj_indices.astype(jnp.int32).reshape(2 * E)
    val = adj_values.astype(jnp.float32)
    xp = jnp.pad(x, ((0, NPAD - N), (0, 0)))
    tab, rd2 = _stage1(xp, W, a2)
    tab32 = lax.bitcast_convert_type(tab.reshape(NPAD, D // 2, 2), jnp.int32)
    acc, sn = _sc_kernel(tab32, rd2, rc, val)
    return _stage3(acc, sn, a1)[:N]


# speedup vs baseline: 1.9402x; 1.9402x over previous
import functools

import jax
import jax.numpy as jnp
from jax import lax
from jax.experimental import pallas as pl
from jax.experimental.pallas import tpu as pltpu
from jax.experimental.pallas import tpu_sc as plsc

N = 10000
E = 320000
D = 128
NC = 2
NS = 16
NW = NC * NS
EPW = E // NW
CHUNK = 80
NCHUNK = EPW // CHUNK
SBC = 25
NSB = NCHUNK // SBC
NBUF = 4
NROUND = (SBC - 1) // NBUF
CVW = 3 * CHUNK
NPAD = 10240
ROWS_PER_TILE = NPAD // NS
SNR = NPAD // D


_B1 = 2048


def _stage1_body(x_ref, w_ref, a2_ref, tab_ref, rd2_ref):
    s = jnp.dot(x_ref[...], w_ref[...], preferred_element_type=jnp.float32)
    rd = jax.nn.sigmoid(
        jnp.dot(s, a2_ref[...], preferred_element_type=jnp.float32))
    tab_ref[...] = rd * s
    r_i = lax.broadcasted_iota(jnp.int32, (_B1 // D, _B1), 0)
    n_i = lax.broadcasted_iota(jnp.int32, (_B1 // D, _B1), 1)
    b1t = (n_i // D == r_i).astype(jnp.float32)
    n2 = lax.broadcasted_iota(jnp.int32, (_B1, D), 0)
    l2 = lax.broadcasted_iota(jnp.int32, (_B1, D), 1)
    sel = (n2 % D == l2).astype(jnp.float32)
    rd2_ref[...] = jnp.dot(b1t, rd * sel,
                           preferred_element_type=jnp.float32)


def _stage1(xp, W, a2):
    return pl.pallas_call(
        _stage1_body,
        grid=(NPAD // _B1,),
        in_specs=[
            pl.BlockSpec((_B1, D), lambda i: (i, 0)),
            pl.BlockSpec((D, D), lambda i: (0, 0)),
            pl.BlockSpec((D, 1), lambda i: (0, 0)),
        ],
        out_specs=[
            pl.BlockSpec((_B1, D), lambda i: (i, 0)),
            pl.BlockSpec((_B1 // D, D), lambda i: (i, 0)),
        ],
        out_shape=[
            jax.ShapeDtypeStruct((NPAD, D), jnp.float32),
            jax.ShapeDtypeStruct((SNR, D), jnp.float32),
        ],
    )(xp, W, a2)


def _sc_body(tab_hbm, rd2_hbm, rc_hbm, val_hbm, out_hbm, sn_hbm,
             col_s, row_s, val_s, rowv0, rowv1, rowv2, rowv3,
             rows0, rows1, rows2, rows3,
             acc, st0, sg0, sg1, sg2, sg3, ss0, ss1, ss2, ss3):
    c = lax.axis_index("c")
    s = lax.axis_index("s")
    w = s * NC + c

    rows_b = [rows0, rows1, rows2, rows3]
    rowv_b = [rowv0, rowv1, rowv2, rowv3]
    sg = [sg0, sg1, sg2, sg3]
    ss = [ss0, ss1, ss2, ss3]

    zf16 = jnp.zeros((16,), jnp.float32)

    def zero_body(i, carry):
        for j in range(D // 16):
            rows0[i, pl.ds(j * 16, 16)] = zf16
        return carry
    lax.fori_loop(0, CHUNK, zero_body, 0, unroll=False)
    for b in range(ROWS_PER_TILE // CHUNK):
        pltpu.async_copy(
            rows0, acc.at[pl.ds(s * ROWS_PER_TILE + b * CHUNK, CHUNK)], ss0)
    for b in range(ROWS_PER_TILE // CHUNK):
        pltpu.make_async_copy(
            rows0, acc.at[pl.ds(s * ROWS_PER_TILE + b * CHUNK, CHUNK)],
            ss0).wait()
    plsc.subcore_barrier()

    def start_gather(kk, b):
        pltpu.async_copy(tab_hbm.at[col_s.at[pl.ds(kk * CHUNK, CHUNK)]],
                         rows_b[b], sg[b])

    def wait_gather(kk, b):
        pltpu.make_async_copy(tab_hbm.at[col_s.at[pl.ds(kk * CHUNK, CHUNK)]],
                              rows_b[b], sg[b]).wait()

    def start_scatter(b):
        pltpu.async_copy(rows_b[b], acc.at[rowv_b[b]], ss[b], add=True)

    def wait_scatter(b):
        pltpu.make_async_copy(rows_b[b], acc.at[rowv_b[b]], ss[b]).wait()

    def process(kk, b):
        def group_body(g, carry2):
            base = kk * CHUNK + g * 16
            row16 = row_s[pl.ds(base, 16)]
            val16 = val_s[pl.ds(base, 16)]
            rowv_b[b][pl.ds(g * 16, 16)] = row16
            for l in range(16):
                v = val16[l]
                e = g * 16 + l
                for j in range(D // 16):
                    sl = pl.ds(j * 16, 16)
                    rows_b[b][e, sl] = rows_b[b][e, sl] * v
            return carry2

        lax.fori_loop(0, CHUNK // 16, group_body, 0, unroll=False)

    def stage_cv(sb):
        base = w * EPW + sb * (SBC * CHUNK)
        pltpu.async_copy(rc_hbm.at[pl.ds(E + base, SBC * CHUNK)], col_s, st0)
        pltpu.async_copy(rc_hbm.at[pl.ds(base, SBC * CHUNK)], row_s, st0)
        pltpu.async_copy(val_hbm.at[pl.ds(base, SBC * CHUNK)], val_s, st0)
        pltpu.make_async_copy(rc_hbm.at[pl.ds(E + base, SBC * CHUNK)],
                              col_s, st0).wait()
        pltpu.make_async_copy(rc_hbm.at[pl.ds(base, SBC * CHUNK)],
                              row_s, st0).wait()
        pltpu.make_async_copy(val_hbm.at[pl.ds(base, SBC * CHUNK)],
                              val_s, st0).wait()

    def sb_body(sb, carry):
        stage_cv(sb)
        start_gather(0, 0)
        start_gather(1, 1)

        def round_body(r, carry2):
            for b in range(NBUF):
                k = r * NBUF + b
                bp = (b + 2) % NBUF

                @pl.when(k >= 2)
                def _():
                    wait_scatter(bp)

                @pl.when(k <= SBC - 3)
                def _():
                    start_gather(k + 2, bp)

                wait_gather(k, b)
                process(k, b)
                start_scatter(b)
            return carry2

        lax.fori_loop(0, NROUND, round_body, 0, unroll=False)

        wait_scatter(2)
        wait_gather(SBC - 1, 0)
        process(SBC - 1, 0)
        start_scatter(0)
        wait_scatter(3)
        wait_scatter(0)
        return carry

    lax.fori_loop(0, NSB, sb_body, 0, unroll=False)

    def zero_sn(i, carry):
        for j in range(D // 16):
            rows1[i, pl.ds(j * 16, 16)] = zf16
        return carry
    lax.fori_loop(0, CHUNK, zero_sn, 0, unroll=False)
    pltpu.sync_copy(rd2_hbm, rows0)

    def sb2_body(sb, carry):
        stage_cv(sb)

        def chunk2(kk, carry2):
            def group2(g, carry3):
                base = kk * CHUNK + g * 16
                col16 = col_s[pl.ds(base, 16)]
                row16 = row_s[pl.ds(base, 16)]
                val16 = val_s[pl.ds(base, 16)]
                rd16 = plsc.load_gather(
                    rows0,
                    [lax.shift_right_logical(col16, 7),
                     lax.bitwise_and(col16, 127)])
                plsc.addupdate_scatter(
                    rows1,
                    [lax.shift_right_logical(row16, 7),
                     lax.bitwise_and(row16, 127)],
                    val16 * rd16)
                return carry3

            return lax.fori_loop(0, CHUNK // 16, group2, carry2,
                                 unroll=False)

        lax.fori_loop(0, SBC, chunk2, 0, unroll=False)
        return carry

    lax.fori_loop(0, NSB, sb2_body, 0, unroll=False)

    pltpu.sync_copy(rows1, sn_hbm.at[w])
    plsc.subcore_barrier()

    pltpu.sync_copy(acc.at[pl.ds(s * ROWS_PER_TILE, ROWS_PER_TILE)],
                    out_hbm.at[c, pl.ds(s * ROWS_PER_TILE, ROWS_PER_TILE)])


_sc_kernel = functools.partial(
    pl.kernel,
    out_type=[
        jax.ShapeDtypeStruct((NC, NPAD, D), jnp.float32),
        jax.ShapeDtypeStruct((NW, SNR, D), jnp.float32),
    ],
    mesh=plsc.VectorSubcoreMesh(core_axis_name="c", subcore_axis_name="s"),
    compiler_params=pltpu.CompilerParams(needs_layout_passes=False),
    scratch_types=[
        pltpu.VMEM((SBC * CHUNK,), jnp.int32),
        pltpu.VMEM((SBC * CHUNK,), jnp.int32),
        pltpu.VMEM((SBC * CHUNK,), jnp.float32),
        pltpu.VMEM((CHUNK,), jnp.int32),
        pltpu.VMEM((CHUNK,), jnp.int32),
        pltpu.VMEM((CHUNK,), jnp.int32),
        pltpu.VMEM((CHUNK,), jnp.int32),
        pltpu.VMEM((CHUNK, D), jnp.float32),
        pltpu.VMEM((CHUNK, D), jnp.float32),
        pltpu.VMEM((CHUNK, D), jnp.float32),
        pltpu.VMEM((CHUNK, D), jnp.float32),
        pltpu.VMEM_SHARED((NPAD, D), jnp.float32),
        pltpu.SemaphoreType.DMA,
        pltpu.SemaphoreType.DMA,
        pltpu.SemaphoreType.DMA,
        pltpu.SemaphoreType.DMA,
        pltpu.SemaphoreType.DMA,
        pltpu.SemaphoreType.DMA,
        pltpu.SemaphoreType.DMA,
        pltpu.SemaphoreType.DMA,
        pltpu.SemaphoreType.DMA,
    ],
)(_sc_body)


_B3 = 5120


def _stage3_body(acc_ref, sn_ref, a1_ref, out_ref):
    a = acc_ref[0] + acc_ref[1]
    sn_sum = jnp.sum(sn_ref[...], axis=0)
    rows = lax.broadcasted_iota(jnp.int32, (_B3, _B3 // D), 0)
    cols = lax.broadcasted_iota(jnp.int32, (_B3, _B3 // D), 1)
    b1 = (rows // D == cols).astype(jnp.float32)
    tmp = jnp.dot(b1, sn_sum, preferred_element_type=jnp.float32)
    rows2 = lax.broadcasted_iota(jnp.int32, (_B3, D), 0)
    cols2 = lax.broadcasted_iota(jnp.int32, (_B3, D), 1)
    sel = (rows2 % D == cols2).astype(jnp.float32)
    snb = jnp.dot(tmp * sel, jnp.ones((D, D), jnp.float32),
                  preferred_element_type=jnp.float32)
    out = a / snb
    l_d = jax.nn.softplus(
        jnp.dot(out, a1_ref[...], preferred_element_type=jnp.float32))
    out = jnp.maximum(out, 0.0) + l_d * jnp.minimum(out, 0.0)
    out_ref[...] = jnp.where(out > 0, out,
                             jnp.exp(jnp.minimum(out, 0.0)) - 1.0)


def _stage3(acc, sn, a1):
    return pl.pallas_call(
        _stage3_body,
        grid=(NPAD // _B3,),
        in_specs=[
            pl.BlockSpec((NC, _B3, D), lambda i: (0, i, 0)),
            pl.BlockSpec((NW, _B3 // D, D), lambda i: (0, i, 0)),
            pl.BlockSpec((D, 1), lambda i: (0, 0)),
        ],
        out_specs=pl.BlockSpec((_B3, D), lambda i: (i, 0)),
        out_shape=jax.ShapeDtypeStruct((NPAD, D), jnp.float32),
    )(acc, sn, a1)


def kernel(x, adj_indices, adj_values, W, a1, a2):
    rc = adj_indices.astype(jnp.int32).reshape(2 * E)
    val = adj_values.astype(jnp.float32)
    xp = jnp.pad(x, ((0, NPAD - N), (0, 0)))
    tab, rd2 = _stage1(xp, W, a2)
    acc, sn = _sc_kernel(tab, rd2, rc, val)
    return _stage3(acc, sn, a1)[:N]
